# Initial kernel scaffold; baseline (speedup 1.0000x reference)
import jax
import jax.numpy as jnp
from jax.experimental import pallas as pl


def kernel(midi_tokens, W1, b1, W2, b2):
    raise NotImplementedError("write your pallas kernel here")


# speedup vs baseline: 2.3352x; 2.3352x over previous
import functools

import jax
import jax.numpy as jnp
from jax import lax
from jax.experimental import pallas as pl
from jax.experimental.pallas import tpu as pltpu
from jax.experimental.pallas import tpu_sc as plsc

B, T = 16384, 200
NBINS = 128
HIDDEN = 256
FEAT = 128

_info = plsc.get_sparse_core_info()
_NC, _NS = _info.num_cores, _info.num_subcores
_NW = _NC * _NS
_SB = 256
_ROWS_PER_W = B // _NW
_NSB = _ROWS_PER_W // _SB
_UNROLL = 8


def _sc_histogram(tokens_flat, zeros_flat):
    mesh = plsc.VectorSubcoreMesh(core_axis_name="c", subcore_axis_name="s")

    @functools.partial(
        pl.kernel,
        mesh=mesh,
        out_type=jax.ShapeDtypeStruct((B * NBINS,), jnp.float32),
        scratch_types=[
            pltpu.VMEM((_SB * T,), jnp.int32),
            pltpu.VMEM((_SB * NBINS,), jnp.float32),
        ],
        compiler_params=pltpu.CompilerParams(needs_layout_passes=False),
    )
    def hist_kernel(tok_hbm, zero_hbm, counts_hbm, tok_v, hist_v):
        wid = lax.axis_index("s") * _NC + lax.axis_index("c")
        lane = lax.iota(jnp.int32, 16)
        ones = jnp.ones((16,), jnp.float32)

        for sb in range(_NSB):
            base = (wid * _NSB + sb) * _SB
            pltpu.sync_copy(tok_hbm.at[pl.ds(base * T, _SB * T)], tok_v)
            pltpu.sync_copy(zero_hbm, hist_v)

            for blk in range(_SB // 16):
                rows = blk * 16 + lane
                tok_base = rows * T
                hist_base = rows * NBINS

                @plsc.parallel_loop(0, T, unroll=_UNROLL)
                def t_body(t):
                    v = plsc.load_gather(tok_v, [tok_base + t])
                    p = jnp.bitwise_and(v, NBINS - 1)
                    plsc.addupdate_scatter(hist_v, [hist_base + p], ones)

            pltpu.sync_copy(hist_v, counts_hbm.at[pl.ds(base * NBINS, _SB * NBINS)])

    return hist_kernel(tokens_flat, zeros_flat)


_R = 512


def _tc_mlp(tokens, counts, harmony, w1p, w1r, w1h, b1, w2, b2):
    def mlp_body(tok_ref, counts_ref, har_ref, w1p_ref, w1r_ref, w1h_ref,
                 b1_ref, w2_ref, b2_ref, out_ref):
        tf = tok_ref[...].astype(jnp.float32)
        mean = jnp.sum(tf, axis=1, keepdims=True) * (1.0 / T)
        d = tf - mean
        var = jnp.sum(d * d, axis=1, keepdims=True) * (1.0 / (T - 1))
        std = jnp.sqrt(var)
        cn = counts_ref[...] * (1.0 / T)
        h = jnp.dot(cn, w1p_ref[...], preferred_element_type=jnp.float32)
        h += mean * w1r_ref[0:1, :]
        h += std * w1r_ref[1:2, :]
        h += jnp.dot(har_ref[...], w1h_ref[...], preferred_element_type=jnp.float32)
        h += b1_ref[...]
        h = jnp.maximum(h, 0.0)
        out_ref[...] = (
            jnp.dot(h, w2_ref[...], preferred_element_type=jnp.float32) + b2_ref[...]
        )

    return pl.pallas_call(
        mlp_body,
        grid=(B // _R,),
        in_specs=[
            pl.BlockSpec((_R, T), lambda i: (i, 0)),
            pl.BlockSpec((_R, NBINS), lambda i: (i, 0)),
            pl.BlockSpec((_R, 12), lambda i: (i, 0)),
            pl.BlockSpec((NBINS, HIDDEN), lambda i: (0, 0)),
            pl.BlockSpec((2, HIDDEN), lambda i: (0, 0)),
            pl.BlockSpec((12, HIDDEN), lambda i: (0, 0)),
            pl.BlockSpec((1, HIDDEN), lambda i: (0, 0)),
            pl.BlockSpec((HIDDEN, FEAT), lambda i: (0, 0)),
            pl.BlockSpec((1, FEAT), lambda i: (0, 0)),
        ],
        out_specs=pl.BlockSpec((_R, FEAT), lambda i: (i, 0)),
        out_shape=jax.ShapeDtypeStruct((B, FEAT), jnp.float32),
    )(tokens, counts, harmony, w1p, w1r, w1h, b1, w2, b2)


def kernel(midi_tokens, W1, b1, W2, b2):
    zeros_flat = jnp.zeros((_SB * NBINS,), jnp.float32)
    counts_flat = _sc_histogram(midi_tokens.reshape(B * T), zeros_flat)
    counts = counts_flat.reshape(B, NBINS)
    harmony = jax.random.uniform(jax.random.key(42), (B, 12), dtype=jnp.float32)
    return _tc_mlp(
        midi_tokens,
        counts,
        harmony,
        W1[:NBINS],
        W1[NBINS:NBINS + 2],
        W1[NBINS + 10:NBINS + 22],
        b1.reshape(1, HIDDEN),
        W2,
        b2.reshape(1, FEAT),
    )
